# baseline (speedup 1.0000x reference)
import functools

import jax
import jax.numpy as jnp
from jax import lax
from jax.experimental import pallas as pl
from jax.experimental.pallas import tpu as pltpu
from jax.experimental.pallas import tpu_sc as plsc

VOC = 1_000_000
DIM = 32
ROWS = 16384
COLS = 26
PAD = 128
NC = 2
NS = 16
NW = NC * NS
BPW = ROWS // NW
CHB = 256
NCHUNK = COLS * (BPW // CHB)
NBUF = 3
PER_W = COLS * BPW
VBLK = 2048


@functools.partial(
    pl.pallas_call,
    out_shape=jax.ShapeDtypeStruct((VOC, PAD), jnp.float32),
    grid=((VOC + VBLK - 1) // VBLK,),
    in_specs=[pl.BlockSpec((DIM, VBLK), lambda i: (0, i))],
    out_specs=pl.BlockSpec((VBLK, PAD), lambda i: (i, 0)),
)
def _tc_pad(tt_ref, o_ref):
    o_ref[:, :DIM] = lax.dot_general(
        tt_ref[...], jnp.eye(DIM, dtype=jnp.float32),
        (((0,), (0,)), ((), ())),
        preferred_element_type=jnp.float32,
        precision=lax.Precision.HIGHEST)


@functools.partial(
    pl.kernel,
    out_type=jax.ShapeDtypeStruct((COLS, DIM, ROWS), jnp.float32),
    mesh=plsc.VectorSubcoreMesh(core_axis_name="c", subcore_axis_name="s"),
    scratch_types=(
        [pltpu.VMEM((PER_W,), jnp.int32)]
        + [pltpu.VMEM((CHB, PAD), jnp.float32) for _ in range(NBUF)]
        + [pltpu.VMEM((DIM, CHB), jnp.float32)]
        + [pltpu.SemaphoreType.DMA for _ in range(NBUF)]
        + [pltpu.SemaphoreType.DMA]
    ),
    compiler_params=pltpu.CompilerParams(
        use_tc_tiling_on_sc=True, needs_layout_passes=False,
        disable_bounds_checks=True),
)
def _sc_gather(xt_hbm, tp_hbm, out_hbm, idx_v, *bufs):
    wid = lax.axis_index("s") * NC + lax.axis_index("c")
    b0 = wid * BPW

    wide = bufs[:NBUF]
    comp = bufs[NBUF]
    sems = bufs[NBUF + 1:2 * NBUF + 1]
    isem = bufs[2 * NBUF + 1]

    for s in range(COLS):
        pltpu.async_copy(
            xt_hbm.at[s, pl.ds(b0, BPW)],
            idx_v.at[pl.ds(s * BPW, BPW)], isem)
    for s in range(COLS):
        pltpu.make_async_copy(
            xt_hbm.at[s, pl.ds(b0, BPW)],
            idx_v.at[pl.ds(s * BPW, BPW)], isem).wait()

    iota = lax.iota(jnp.int32, 16)

    def start_gather(c, b):
        pltpu.async_copy(
            tp_hbm.at[idx_v.at[pl.ds(c * CHB, CHB)]], wide[b], sems[b])

    def wait_gather(c, b):
        pltpu.make_async_copy(
            tp_hbm.at[idx_v.at[pl.ds(c * CHB, CHB)]], wide[b],
            sems[b]).wait()

    def extract(b):
        def _grp(g, carry):
            kvec = g * 16 + iota
            for d in range(DIM):
                val = plsc.load_gather(
                    wide[b], [kvec, jnp.full((16,), d, jnp.int32)])
                comp[d, pl.ds(g * 16, 16)] = val
            return carry
        lax.fori_loop(0, CHB // 16, _grp, 0)

    def writeback(c, b):
        s = c >> 1
        bb = b0 + (c & 1) * CHB
        pltpu.sync_copy(comp, out_hbm.at[s, :, pl.ds(bb, CHB)])

    def step(c, b):
        wait_gather(c, b)
        extract(b)
        writeback(c, b)

    for b in range(NBUF):
        start_gather(b, b)

    def body(i, carry):
        for b in range(NBUF):
            c = i * NBUF + b
            step(c, b)
            start_gather(c + NBUF, b)
        return carry

    steady = (NCHUNK - NBUF) // NBUF
    lax.fori_loop(0, steady, body, 0)
    for c in range(steady * NBUF, NCHUNK):
        step(c, c % NBUF)
        if c + NBUF < NCHUNK:
            start_gather(c + NBUF, c % NBUF)


def kernel(x, table):
    tp = _tc_pad(table.T)
    out = _sc_gather(x.T, tp)
    return out.transpose(2, 0, 1)


# speedup vs baseline: 1.1807x; 1.1807x over previous
import functools

import jax
import jax.numpy as jnp
from jax import lax
from jax.experimental import pallas as pl
from jax.experimental.pallas import tpu as pltpu
from jax.experimental.pallas import tpu_sc as plsc

VOC = 1_000_000
DIM = 32
ROWS = 16384
COLS = 26
NC = 2
NS = 16
NW = NC * NS
BPW = ROWS // NW
NBUF = 4


@functools.partial(
    pl.kernel,
    out_type=jax.ShapeDtypeStruct((ROWS, COLS, DIM), jnp.float32),
    mesh=plsc.VectorSubcoreMesh(core_axis_name="c", subcore_axis_name="s"),
    scratch_types=(
        [pltpu.VMEM((COLS, BPW), jnp.int32)]
        + [pltpu.VMEM((BPW, DIM), jnp.float32) for _ in range(NBUF)]
        + [pltpu.SemaphoreType.DMA for _ in range(NBUF)]
    ),
    compiler_params=pltpu.CompilerParams(use_tc_tiling_on_sc=False),
)
def _sc_gather(xt_hbm, table_hbm, out_hbm, idx_v, *bufs):
    wid = lax.axis_index("s") * NC + lax.axis_index("c")
    b0 = wid * BPW

    pltpu.sync_copy(xt_hbm.at[:, pl.ds(b0, BPW)], idx_v)

    rows = bufs[:NBUF]
    sems = bufs[NBUF:]

    for b in range(NBUF):
        pltpu.async_copy(table_hbm.at[idx_v.at[b]], rows[b], sems[b])

    def step(j, b):
        pltpu.make_async_copy(
            table_hbm.at[idx_v.at[j]], rows[b], sems[b]).wait()
        pltpu.sync_copy(rows[b], out_hbm.at[pl.ds(b0, BPW), j])

    def body(i, carry):
        for b in range(NBUF):
            j = i * NBUF + b
            step(j, b)
            pltpu.async_copy(
                table_hbm.at[idx_v.at[j + NBUF]], rows[b], sems[b])
        return carry

    steady = (COLS - NBUF) // NBUF
    lax.fori_loop(0, steady, body, 0)
    for k in range(steady * NBUF, COLS - NBUF):
        b = k % NBUF
        step(k, b)
        pltpu.async_copy(
            table_hbm.at[idx_v.at[k + NBUF]], rows[b], sems[b])
    for j in range(COLS - NBUF, COLS):
        step(j, j % NBUF)


def kernel(x, table):
    return _sc_gather(x.T, table)
